# baseline (speedup 1.0000x reference)
import jax
import jax.numpy as jnp
from jax.experimental import pallas as pl
from jax.experimental.pallas import tpu as pltpu

_N = 4608
_C = 256
_BM = 256
_BN = 256
_NB = _N // _BM
_NCH = _N // _BN
_NCLS = 128


def _snnl_rows_kernel(xi_ref, xall_ref, yrow_ref, oh_ref, out_ref, e_scr):
    i = pl.program_id(0)
    i0 = i * _BM
    xi = xi_ref[...]
    sq_i = jnp.sum(xi * xi, axis=1, keepdims=True)
    ones8 = jnp.ones((8, _C), dtype=jnp.float32)
    sacc = jnp.zeros((_BM, _NCLS), dtype=jnp.float32)
    for j in range(_NCH):
        j0 = j * _BN
        xj = xall_ref[j0:j0 + _BN, :]
        dotv = jax.lax.dot_general(
            xi, xj, (((1,), (1,)), ((), ())),
            preferred_element_type=jnp.float32)
        sq_j = jax.lax.dot_general(
            ones8, xj * xj, (((1,), (1,)), ((), ())),
            preferred_element_type=jnp.float32)[0:1, :]
        d = jnp.maximum((sq_i + sq_j) - 2.0 * dotv, 0.0)
        e_scr[...] = jnp.exp(-d)

        @pl.when(j0 == i0)
        def _zero_diag():
            rr = jax.lax.broadcasted_iota(jnp.int32, (_BM, _BN), 0)
            cc = jax.lax.broadcasted_iota(jnp.int32, (_BM, _BN), 1)
            e_scr[...] = jnp.where(rr == cc, 0.0, e_scr[...])

        sacc = sacc + jax.lax.dot_general(
            e_scr[...], oh_ref[j0:j0 + _BN, :], (((1,), (0,)), ((), ())),
            preferred_element_type=jnp.float32)
    den = jnp.sum(sacc, axis=1, keepdims=True)
    num = jnp.take_along_axis(sacc, yrow_ref[...], axis=1)
    out_ref[...] = jnp.log(num / den)


def _mean_kernel(v_ref, o_ref):
    s = jnp.sum(v_ref[...], axis=1, keepdims=True)
    o_ref[...] = s * (-1.0 / _N)


def kernel(labels, outputs, features, train_step, epoch):
    y = labels[:, ::8, ::8].reshape(-1).astype(jnp.int32)
    x = features.reshape(-1, _C).astype(jnp.float32)
    oh = (y[:, None] == jnp.arange(_NCLS, dtype=jnp.int32)[None, :])
    oh = oh.astype(jnp.float32)
    logr = pl.pallas_call(
        _snnl_rows_kernel,
        grid=(_NB,),
        in_specs=[
            pl.BlockSpec((_BM, _C), lambda i: (i, 0)),
            pl.BlockSpec((_N, _C), lambda i: (0, 0)),
            pl.BlockSpec((_BM, 1), lambda i: (i, 0)),
            pl.BlockSpec((_N, _NCLS), lambda i: (0, 0)),
        ],
        out_specs=pl.BlockSpec((_BM, 1), lambda i: (i, 0)),
        out_shape=jax.ShapeDtypeStruct((_N, 1), jnp.float32),
        scratch_shapes=[pltpu.VMEM((_BM, _BN), jnp.float32)],
        compiler_params=pltpu.CompilerParams(
            dimension_semantics=("parallel",),
        ),
        name="snnl_rows",
    )(x, x, y.reshape(_N, 1), oh)
    loss = pl.pallas_call(
        _mean_kernel,
        out_shape=jax.ShapeDtypeStruct((1, 1), jnp.float32),
        name="snnl_mean",
    )(logr.reshape(1, _N))
    return loss.reshape(())


# speedup vs baseline: 1.3982x; 1.3982x over previous
import jax
import jax.numpy as jnp
from jax.experimental import pallas as pl
from jax.experimental.pallas import tpu as pltpu

_N = 4608
_C = 256
_BM = 256
_BN = 256
_NB = _N // _BM
_NCH = _N // _BN
_NCLS = 128


def _snnl_rows_kernel(xi_ref, xall_ref, yrow_ref, oh_ref, out_ref):
    i = pl.program_id(0)
    i0 = i * _BM
    xi = xi_ref[...]
    sq_i = jnp.sum(xi * xi, axis=1, keepdims=True)
    rc = (jax.lax.broadcasted_iota(jnp.int32, (_BM, _BN), 0) + i0
          - jax.lax.broadcasted_iota(jnp.int32, (_BM, _BN), 1))
    ones8 = jnp.ones((8, _C), dtype=jnp.float32)
    sacc = jnp.zeros((_BM, _NCLS), dtype=jnp.float32)
    for j in range(_NCH):
        j0 = j * _BN
        xj = xall_ref[j0:j0 + _BN, :]
        dotv = jax.lax.dot_general(
            xi, xj, (((1,), (1,)), ((), ())),
            preferred_element_type=jnp.float32)
        sq_j = jax.lax.dot_general(
            ones8, xj * xj, (((1,), (1,)), ((), ())),
            preferred_element_type=jnp.float32)[0:1, :]
        d = jnp.maximum((sq_i + sq_j) - 2.0 * dotv, 0.0)
        e = jnp.where(rc == j0, 0.0, jnp.exp(-d))
        sacc = sacc + jax.lax.dot_general(
            e, oh_ref[j0:j0 + _BN, :], (((1,), (0,)), ((), ())),
            preferred_element_type=jnp.float32)
    den = jnp.sum(sacc, axis=1, keepdims=True)
    num = jnp.take_along_axis(sacc, yrow_ref[...], axis=1)
    out_ref[...] = jnp.log(num / den)


def _mean_kernel(v_ref, o_ref):
    s = jnp.sum(v_ref[...], axis=1, keepdims=True)
    o_ref[...] = s * (-1.0 / _N)


def kernel(labels, outputs, features, train_step, epoch):
    y = labels[:, ::8, ::8].reshape(-1).astype(jnp.int32)
    x = features.reshape(-1, _C).astype(jnp.float32)
    oh = (y[:, None] == jnp.arange(_NCLS, dtype=jnp.int32)[None, :])
    oh = oh.astype(jnp.float32)
    logr = pl.pallas_call(
        _snnl_rows_kernel,
        grid=(_NB,),
        in_specs=[
            pl.BlockSpec((_BM, _C), lambda i: (i, 0)),
            pl.BlockSpec((_N, _C), lambda i: (0, 0)),
            pl.BlockSpec((_BM, 1), lambda i: (i, 0)),
            pl.BlockSpec((_N, _NCLS), lambda i: (0, 0)),
        ],
        out_specs=pl.BlockSpec((_BM, 1), lambda i: (i, 0)),
        out_shape=jax.ShapeDtypeStruct((_N, 1), jnp.float32),
        compiler_params=pltpu.CompilerParams(
            dimension_semantics=("parallel",),
        ),
        name="snnl_rows",
    )(x, x, y.reshape(_N, 1), oh)
    loss = pl.pallas_call(
        _mean_kernel,
        out_shape=jax.ShapeDtypeStruct((1, 1), jnp.float32),
        name="snnl_mean",
    )(logr.reshape(1, _N))
    return loss.reshape(())


# speedup vs baseline: 1.4492x; 1.0365x over previous
import jax
import jax.numpy as jnp
from jax.experimental import pallas as pl
from jax.experimental.pallas import tpu as pltpu

_N = 4608
_C = 256
_BM = 256
_BN = 256
_NB = _N // _BM
_NCH = _N // _BN
_NCLS = 128


def _snnl_kernel(xi_ref, xall_ref, yrow_ref, oh_ref, out_ref, acc_ref):
    i = pl.program_id(0)
    i0 = i * _BM
    xi = xi_ref[...]
    sq_i = jnp.sum(xi * xi, axis=1, keepdims=True)
    rc = (jax.lax.broadcasted_iota(jnp.int32, (_BM, _BN), 0) + i0
          - jax.lax.broadcasted_iota(jnp.int32, (_BM, _BN), 1))
    ones8 = jnp.ones((8, _C), dtype=jnp.float32)
    sacc = jnp.zeros((_BM, _NCLS), dtype=jnp.float32)
    for j in range(_NCH):
        j0 = j * _BN
        xj = xall_ref[j0:j0 + _BN, :]
        dotv = jax.lax.dot_general(
            xi, xj, (((1,), (1,)), ((), ())),
            preferred_element_type=jnp.float32)
        sq_j = jax.lax.dot_general(
            ones8, xj * xj, (((1,), (1,)), ((), ())),
            preferred_element_type=jnp.float32)[0:1, :]
        d = jnp.maximum((sq_i + sq_j) - 2.0 * dotv, 0.0)
        e = jnp.where(rc == j0, 0.0, jnp.exp(-d))
        sacc = sacc + jax.lax.dot_general(
            e, oh_ref[j0:j0 + _BN, :], (((1,), (0,)), ((), ())),
            preferred_element_type=jnp.float32)
    den = jnp.sum(sacc, axis=1, keepdims=True)
    num = jnp.take_along_axis(sacc, yrow_ref[...], axis=1)
    r = jnp.log(num / den)
    part = jnp.sum(jnp.broadcast_to(r, (_BM, _NCLS)), axis=0, keepdims=True)

    @pl.when(i == 0)
    def _init():
        acc_ref[...] = jnp.zeros_like(acc_ref)

    acc_ref[...] += part

    @pl.when(i == _NB - 1)
    def _fin():
        out_ref[...] = acc_ref[0:1, 0:1] * (-1.0 / _N)


def kernel(labels, outputs, features, train_step, epoch):
    y = labels[:, ::8, ::8].reshape(-1).astype(jnp.int32)
    x = features.reshape(-1, _C).astype(jnp.float32)
    oh = (y[:, None] == jnp.arange(_NCLS, dtype=jnp.int32)[None, :])
    oh = oh.astype(jnp.float32)
    loss = pl.pallas_call(
        _snnl_kernel,
        grid=(_NB,),
        in_specs=[
            pl.BlockSpec((_BM, _C), lambda i: (i, 0)),
            pl.BlockSpec((_N, _C), lambda i: (0, 0)),
            pl.BlockSpec((_BM, 1), lambda i: (i, 0)),
            pl.BlockSpec((_N, _NCLS), lambda i: (0, 0)),
        ],
        out_specs=pl.BlockSpec((1, 1), lambda i: (0, 0)),
        out_shape=jax.ShapeDtypeStruct((1, 1), jnp.float32),
        scratch_shapes=[pltpu.VMEM((1, _NCLS), jnp.float32)],
        compiler_params=pltpu.CompilerParams(
            dimension_semantics=("arbitrary",),
        ),
        name="snnl_fused",
    )(x, x, y.reshape(_N, 1), oh)
    return loss.reshape(())
